# Initial kernel scaffold; baseline (speedup 1.0000x reference)
import functools

import jax
import jax.numpy as jnp
from jax import lax
from jax.experimental import pallas as pl
from jax.experimental.pallas import tpu as pltpu
from jax.experimental.pallas import tpu_sc as plsc

N = 10000
D_IN, D_H, D_OUT = 128, 128, 64
E = 320000

N_PAD = 10240
NTILES = 16
NCORES = 2
NW = NCORES * NTILES
CH = 128
EPW = 10112
E_PAD = NW * EPW
NCH = EPW // CH
RPT = N_PAD // NTILES
DEGW = 16

BLK = 512
GRID = N_PAD // BLK


def _fill(ref, width, value):
    per_row = width // 16

    def body(i, _):
        r = i // per_row
        col = (i % per_row) * 16
        ref[r, pl.ds(col, 16)] = jnp.full((16,), value, jnp.float32)
        return 0

    lax.fori_loop(0, ref.shape[0] * per_row, body, 0)


def _make_agg(D, with_deg):
    mesh = plsc.VectorSubcoreMesh(core_axis_name="c", subcore_axis_name="s")
    out_type = [jax.ShapeDtypeStruct((NCORES, N_PAD, D), jnp.float32)]
    scratch = [
        pltpu.VMEM((CH,), jnp.int32),
        pltpu.VMEM((CH,), jnp.int32),
        pltpu.VMEM((CH, D), jnp.float32),
        pltpu.VMEM((CH, D), jnp.float32),
        pltpu.VMEM_SHARED((N_PAD, D), jnp.float32),
        pltpu.SemaphoreType.DMA,
    ]
    if with_deg:
        out_type.append(jax.ShapeDtypeStruct((NCORES, N_PAD, DEGW), jnp.float32))
        scratch += [
            pltpu.VMEM((CH, DEGW), jnp.float32),
            pltpu.VMEM((CH, DEGW), jnp.float32),
            pltpu.VMEM_SHARED((N_PAD, DEGW), jnp.float32),
        ]

    def body(p_hbm, src_hbm, dst_hbm, *rest):
        if with_deg:
            (acc_out, deg_out, sidx, didx, rows, zbuf, acc, sem,
             ones, z16, dacc) = rest
        else:
            acc_out, sidx, didx, rows, zbuf, acc, sem = rest
        c = lax.axis_index("c")
        s = lax.axis_index("s")
        wid = c * NTILES + s

        _fill(zbuf, D, 0.0)
        if with_deg:
            _fill(z16, DEGW, 0.0)
            _fill(ones, DEGW, 1.0)

        base_r = s * RPT
        for j in range(RPT // CH):
            pltpu.sync_copy(zbuf, acc.at[pl.ds(base_r + j * CH, CH)])
            if with_deg:
                pltpu.sync_copy(z16, dacc.at[pl.ds(base_r + j * CH, CH)])
        plsc.subcore_barrier()

        ebase = wid * EPW

        def step(i, _):
            off = ebase + i * CH
            pltpu.sync_copy(src_hbm.at[pl.ds(off, CH)], sidx)
            pltpu.sync_copy(dst_hbm.at[pl.ds(off, CH)], didx)
            pltpu.async_copy(p_hbm.at[sidx], rows, sem).wait()
            pltpu.sync_copy(rows, acc.at[didx], add=True)
            if with_deg:
                pltpu.sync_copy(ones, dacc.at[didx], add=True)
            return 0

        lax.fori_loop(0, NCH, step, 0)
        plsc.subcore_barrier()

        for j in range(RPT // CH):
            r0 = base_r + j * CH
            pltpu.sync_copy(acc.at[pl.ds(r0, CH)], acc_out.at[c, pl.ds(r0, CH)])
            if with_deg:
                pltpu.sync_copy(dacc.at[pl.ds(r0, CH)],
                                deg_out.at[c, pl.ds(r0, CH)])

    return pl.kernel(body, out_type=tuple(out_type), mesh=mesh,
                     scratch_types=tuple(scratch))


_agg_deg = _make_agg(D_H, True)
_agg_h = _make_agg(D_H, False)
_agg_o = _make_agg(D_OUT, False)


def _pre_body(x_ref, wl_ref, wr_ref, bl_ref, p_ref, r_ref):
    h = x_ref[...]
    p_ref[...] = jnp.dot(h, wl_ref[...], preferred_element_type=jnp.float32)
    r_ref[...] = (jnp.dot(h, wr_ref[...], preferred_element_type=jnp.float32)
                  + bl_ref[...])


def _pre(x, wlT, wrT, bl):
    d_in, d_o = wlT.shape
    return pl.pallas_call(
        _pre_body,
        grid=(GRID,),
        in_specs=[
            pl.BlockSpec((BLK, d_in), lambda i: (i, 0)),
            pl.BlockSpec((d_in, d_o), lambda i: (0, 0)),
            pl.BlockSpec((d_in, d_o), lambda i: (0, 0)),
            pl.BlockSpec((1, d_o), lambda i: (0, 0)),
        ],
        out_specs=[
            pl.BlockSpec((BLK, d_o), lambda i: (i, 0)),
            pl.BlockSpec((BLK, d_o), lambda i: (i, 0)),
        ],
        out_shape=[
            jax.ShapeDtypeStruct((N_PAD, d_o), jnp.float32),
            jax.ShapeDtypeStruct((N_PAD, d_o), jnp.float32),
        ],
    )(x, wlT, wrT, bl)


def _combine(acc_ref, dacc_ref, r_ref):
    deg = dacc_ref[0, :, 0:1] + dacc_ref[1, :, 0:1]
    mean = (acc_ref[0] + acc_ref[1]) / jnp.maximum(deg, 1.0)
    return mean + r_ref[...]


def _mid_body(acc_ref, dacc_ref, r_ref, wl_ref, wr_ref, bl_ref, p_ref, rn_ref):
    h = jnp.maximum(_combine(acc_ref, dacc_ref, r_ref), 0.0)
    p_ref[...] = jnp.dot(h, wl_ref[...], preferred_element_type=jnp.float32)
    rn_ref[...] = (jnp.dot(h, wr_ref[...], preferred_element_type=jnp.float32)
                   + bl_ref[...])


def _mid(acc, dacc, r, wlT, wrT, bl):
    d, d_o = wlT.shape
    return pl.pallas_call(
        _mid_body,
        grid=(GRID,),
        in_specs=[
            pl.BlockSpec((NCORES, BLK, d), lambda i: (0, i, 0)),
            pl.BlockSpec((NCORES, BLK, DEGW), lambda i: (0, i, 0)),
            pl.BlockSpec((BLK, d), lambda i: (i, 0)),
            pl.BlockSpec((d, d_o), lambda i: (0, 0)),
            pl.BlockSpec((d, d_o), lambda i: (0, 0)),
            pl.BlockSpec((1, d_o), lambda i: (0, 0)),
        ],
        out_specs=[
            pl.BlockSpec((BLK, d_o), lambda i: (i, 0)),
            pl.BlockSpec((BLK, d_o), lambda i: (i, 0)),
        ],
        out_shape=[
            jax.ShapeDtypeStruct((N_PAD, d_o), jnp.float32),
            jax.ShapeDtypeStruct((N_PAD, d_o), jnp.float32),
        ],
    )(acc, dacc, r, wlT, wrT, bl)


def _final_body(acc_ref, dacc_ref, r_ref, o_ref):
    z = _combine(acc_ref, dacc_ref, r_ref)
    m = jnp.max(z, axis=1, keepdims=True)
    ez = jnp.exp(z - m)
    lse = jnp.log(jnp.sum(ez, axis=1, keepdims=True)) + m
    o_ref[...] = z - lse


def _final(acc, dacc, r):
    d = r.shape[1]
    return pl.pallas_call(
        _final_body,
        grid=(GRID,),
        in_specs=[
            pl.BlockSpec((NCORES, BLK, d), lambda i: (0, i, 0)),
            pl.BlockSpec((NCORES, BLK, DEGW), lambda i: (0, i, 0)),
            pl.BlockSpec((BLK, d), lambda i: (i, 0)),
        ],
        out_specs=pl.BlockSpec((BLK, d), lambda i: (i, 0)),
        out_shape=jax.ShapeDtypeStruct((N_PAD, d), jnp.float32),
    )(acc, dacc, r)


@jax.jit
def kernel(x, edge_index, Wl1, bl1, Wr1, Wl2, bl2, Wr2, Wl3, bl3, Wr3):
    x_pad = jnp.zeros((N_PAD, D_IN), jnp.float32).at[:N].set(x)
    pad_e = E_PAD - E
    src = jnp.concatenate([edge_index[0], jnp.zeros((pad_e,), jnp.int32)])
    dst = jnp.concatenate([edge_index[1], jnp.full((pad_e,), N, jnp.int32)])

    p1, r1 = _pre(x_pad, Wl1.T, Wr1.T, bl1[None])
    acc1, dacc = _agg_deg(p1, src, dst)
    p2, r2 = _mid(acc1, dacc, r1, Wl2.T, Wr2.T, bl2[None])
    acc2 = _agg_h(p2, src, dst)[0]
    p3, r3 = _mid(acc2, dacc, r2, Wl3.T, Wr3.T, bl3[None])
    acc3 = _agg_o(p3, src, dst)[0]
    out = _final(acc3, dacc, r3)
    return out[:N]


# speedup vs baseline: 2.9755x; 2.9755x over previous
import functools

import jax
import jax.numpy as jnp
from jax import lax
from jax.experimental import pallas as pl
from jax.experimental.pallas import tpu as pltpu
from jax.experimental.pallas import tpu_sc as plsc

N = 10000
D_IN, D_H, D_OUT = 128, 128, 64
E = 320000

N_PAD = 10240
NTILES = 16
NCORES = 2
NW = NCORES * NTILES
CH = 128
EPW = 10112
E_PAD = NW * EPW
NCH = EPW // CH
RPT = N_PAD // NTILES
BLK = 512
GRID = N_PAD // BLK


def _fill(ref, width, value):
    per_row = width // 16

    def body(i, _):
        r = i // per_row
        col = (i % per_row) * 16
        ref[r, pl.ds(col, 16)] = jnp.full((16,), value, jnp.float32)
        return 0

    lax.fori_loop(0, ref.shape[0] * per_row, body, 0)


def _make_agg(D):
    mesh = plsc.VectorSubcoreMesh(core_axis_name="c", subcore_axis_name="s")

    def body(p_hbm, src_hbm, dst_hbm, acc_out, sidx, didx, rows, acc, sem):
        c = lax.axis_index("c")
        s = lax.axis_index("s")
        wid = c * NTILES + s

        _fill(rows, D, 0.0)
        base_r = s * RPT
        for j in range(RPT // CH):
            pltpu.sync_copy(rows, acc.at[pl.ds(base_r + j * CH, CH)])
        plsc.subcore_barrier()

        ebase = wid * EPW

        def step(i, _):
            off = ebase + i * CH
            pltpu.sync_copy(src_hbm.at[pl.ds(off, CH)], sidx)
            pltpu.sync_copy(dst_hbm.at[pl.ds(off, CH)], didx)
            pltpu.async_copy(p_hbm.at[sidx], rows, sem).wait()
            pltpu.sync_copy(rows, acc.at[didx], add=True)
            return 0

        lax.fori_loop(0, NCH, step, 0)
        plsc.subcore_barrier()

        for j in range(RPT // CH):
            r0 = base_r + j * CH
            pltpu.sync_copy(acc.at[pl.ds(r0, CH)], rows)
            pltpu.sync_copy(rows, acc_out.at[pl.ds(c * N_PAD + r0, CH)])

    return pl.kernel(
        body,
        out_type=jax.ShapeDtypeStruct((NCORES * N_PAD, D), jnp.float32),
        mesh=mesh,
        scratch_types=(
            pltpu.VMEM((CH,), jnp.int32),
            pltpu.VMEM((CH,), jnp.int32),
            pltpu.VMEM((CH, D), jnp.float32),
            pltpu.VMEM_SHARED((N_PAD, D), jnp.float32),
            pltpu.SemaphoreType.DMA,
        ))


_agg_h = _make_agg(D_H)


def _pre_body(x_ref, wl_ref, wr_ref, bl_ref, p_ref, r_ref):
    h = x_ref[...]
    p_ref[...] = jnp.dot(h, wl_ref[...], preferred_element_type=jnp.float32)
    r_ref[...] = (jnp.dot(h, wr_ref[...], preferred_element_type=jnp.float32)
                  + bl_ref[...])


def _pre(x, wlT, wrT, bl):
    d_in, d_o = wlT.shape
    return pl.pallas_call(
        _pre_body,
        grid=(GRID,),
        in_specs=[
            pl.BlockSpec((BLK, d_in), lambda i: (i, 0)),
            pl.BlockSpec((d_in, d_o), lambda i: (0, 0)),
            pl.BlockSpec((d_in, d_o), lambda i: (0, 0)),
            pl.BlockSpec((1, d_o), lambda i: (0, 0)),
        ],
        out_specs=[
            pl.BlockSpec((BLK, d_o), lambda i: (i, 0)),
            pl.BlockSpec((BLK, d_o), lambda i: (i, 0)),
        ],
        out_shape=[
            jax.ShapeDtypeStruct((N_PAD, d_o), jnp.float32),
            jax.ShapeDtypeStruct((N_PAD, d_o), jnp.float32),
        ],
    )(x, wlT, wrT, bl)


def _combine(acc_ref, dacc_ref, r_ref):
    deg = dacc_ref[0] + dacc_ref[1]
    mean = (acc_ref[0] + acc_ref[1]) / jnp.maximum(deg, 1.0)
    return mean + r_ref[...]


def _mid_body(acc_ref, dacc_ref, r_ref, wl_ref, wr_ref, bl_ref, p_ref, rn_ref):
    h = jnp.maximum(_combine(acc_ref, dacc_ref, r_ref), 0.0)
    p_ref[...] = jnp.dot(h, wl_ref[...], preferred_element_type=jnp.float32)
    rn_ref[...] = (jnp.dot(h, wr_ref[...], preferred_element_type=jnp.float32)
                   + bl_ref[...])


def _mid(acc, dacc, r, wlT, wrT, bl):
    d, d_po = wlT.shape
    d_ro = wrT.shape[1]
    return pl.pallas_call(
        _mid_body,
        grid=(GRID,),
        in_specs=[
            pl.BlockSpec((NCORES, BLK, d), lambda i: (0, i, 0)),
            pl.BlockSpec((NCORES, BLK, 1), lambda i: (0, i, 0)),
            pl.BlockSpec((BLK, d), lambda i: (i, 0)),
            pl.BlockSpec((d, d_po), lambda i: (0, 0)),
            pl.BlockSpec((d, d_ro), lambda i: (0, 0)),
            pl.BlockSpec((1, d_ro), lambda i: (0, 0)),
        ],
        out_specs=[
            pl.BlockSpec((BLK, d_po), lambda i: (i, 0)),
            pl.BlockSpec((BLK, d_ro), lambda i: (i, 0)),
        ],
        out_shape=[
            jax.ShapeDtypeStruct((N_PAD, d_po), jnp.float32),
            jax.ShapeDtypeStruct((N_PAD, d_ro), jnp.float32),
        ],
    )(acc, dacc, r, wlT, wrT, bl)


def _final_body(acc_ref, dacc_ref, r_ref, o_ref):
    d = r_ref.shape[1]
    deg = dacc_ref[0] + dacc_ref[1]
    mean = (acc_ref[0, :, :d] + acc_ref[1, :, :d]) / jnp.maximum(deg, 1.0)
    z = mean + r_ref[...]
    m = jnp.max(z, axis=1, keepdims=True)
    ez = jnp.exp(z - m)
    lse = jnp.log(jnp.sum(ez, axis=1, keepdims=True)) + m
    o_ref[...] = z - lse


def _final(acc, dacc, r):
    d = r.shape[1]
    return pl.pallas_call(
        _final_body,
        grid=(GRID,),
        in_specs=[
            pl.BlockSpec((NCORES, BLK, acc.shape[2]), lambda i: (0, i, 0)),
            pl.BlockSpec((NCORES, BLK, 1), lambda i: (0, i, 0)),
            pl.BlockSpec((BLK, d), lambda i: (i, 0)),
        ],
        out_specs=pl.BlockSpec((BLK, d), lambda i: (i, 0)),
        out_shape=jax.ShapeDtypeStruct((N_PAD, d), jnp.float32),
    )(acc, dacc, r)


@jax.jit
def kernel(x, edge_index, Wl1, bl1, Wr1, Wl2, bl2, Wr2, Wl3, bl3, Wr3):
    x_pad = jnp.zeros((N_PAD, D_IN), jnp.float32).at[:N].set(x)
    pad_e = E_PAD - E
    src = jnp.concatenate([edge_index[0], jnp.zeros((pad_e,), jnp.int32)])
    dst = jnp.concatenate([edge_index[1], jnp.full((pad_e,), N, jnp.int32)])

    ones_feat = jnp.ones((N_PAD, D_H), jnp.float32)
    dacc = _agg_h(ones_feat, src, dst).reshape(NCORES, N_PAD, D_H)[:, :, :1]

    p1, r1 = _pre(x_pad, Wl1.T, Wr1.T, bl1[None])
    acc1 = _agg_h(p1, src, dst).reshape(NCORES, N_PAD, D_H)
    p2, r2 = _mid(acc1, dacc, r1, Wl2.T, Wr2.T, bl2[None])
    acc2 = _agg_h(p2, src, dst).reshape(NCORES, N_PAD, D_H)
    wl3T_pad = jnp.pad(Wl3.T, ((0, 0), (0, D_H - D_OUT)))
    p3, r3 = _mid(acc2, dacc, r2, wl3T_pad, Wr3.T, bl3[None])
    acc3 = _agg_h(p3, src, dst).reshape(NCORES, N_PAD, D_H)
    out = _final(acc3, dacc, r3)
    return out[:N]


# speedup vs baseline: 3.3769x; 1.1349x over previous
import functools

import jax
import jax.numpy as jnp
from jax import lax
from jax.experimental import pallas as pl
from jax.experimental.pallas import tpu as pltpu
from jax.experimental.pallas import tpu_sc as plsc

N = 10000
D_IN, D_H, D_OUT = 128, 128, 64
E = 320000

N_PAD = 10240
NTILES = 16
NCORES = 2
NW = NCORES * NTILES
CH = 128
NCH = 80
EPW = NCH * CH
E_PAD = NW * EPW
NPAIR = NCH // 2
RPT = N_PAD // NTILES
D_DEG = 128
BLK = 512
GRID = N_PAD // BLK


def _fill(ref, width, value):
    per_row = width // 16

    def body(i, _):
        r = i // per_row
        col = (i % per_row) * 16
        ref[r, pl.ds(col, 16)] = jnp.full((16,), value, jnp.float32)
        return 0

    lax.fori_loop(0, ref.shape[0] * per_row, body, 0)


def _make_agg(D):
    mesh = plsc.VectorSubcoreMesh(core_axis_name="c", subcore_axis_name="s")

    def body(p_hbm, src_hbm, dst_hbm, acc_out,
             sidx_a, didx_a, sidx_b, didx_b, rows_a, rows_b,
             acc, gsem_a, gsem_b):
        c = lax.axis_index("c")
        s = lax.axis_index("s")
        wid = c * NTILES + s

        _fill(rows_a, D, 0.0)
        base_r = s * RPT
        for j in range(RPT // CH):
            pltpu.sync_copy(rows_a, acc.at[pl.ds(base_r + j * CH, CH)])
        plsc.subcore_barrier()

        ebase = wid * EPW

        def load_idx(off, sidx, didx):
            pltpu.sync_copy(src_hbm.at[pl.ds(off, CH)], sidx)
            pltpu.sync_copy(dst_hbm.at[pl.ds(off, CH)], didx)

        load_idx(ebase, sidx_a, didx_a)
        pltpu.async_copy(p_hbm.at[sidx_a], rows_a, gsem_a)
        load_idx(ebase + CH, sidx_b, didx_b)

        def step(j, _):
            i0 = 2 * j
            pltpu.async_copy(p_hbm.at[sidx_b], rows_b, gsem_b)
            pltpu.make_async_copy(p_hbm.at[sidx_a], rows_a, gsem_a).wait()
            pltpu.sync_copy(rows_a, acc.at[didx_a], add=True)
            load_idx(ebase + (i0 + 2) * CH, sidx_a, didx_a)
            pltpu.async_copy(p_hbm.at[sidx_a], rows_a, gsem_a)
            pltpu.make_async_copy(p_hbm.at[sidx_b], rows_b, gsem_b).wait()
            pltpu.sync_copy(rows_b, acc.at[didx_b], add=True)
            load_idx(ebase + (i0 + 3) * CH, sidx_b, didx_b)
            return 0

        lax.fori_loop(0, NPAIR, step, 0)
        pltpu.make_async_copy(p_hbm.at[sidx_a], rows_a, gsem_a).wait()
        plsc.subcore_barrier()

        for j in range(RPT // CH):
            r0 = base_r + j * CH
            pltpu.sync_copy(acc.at[pl.ds(r0, CH)], rows_a)
            pltpu.sync_copy(rows_a, acc_out.at[pl.ds(c * N_PAD + r0, CH)])

    return pl.kernel(
        body,
        out_type=jax.ShapeDtypeStruct((NCORES * N_PAD, D), jnp.float32),
        mesh=mesh,
        scratch_types=(
            pltpu.VMEM((CH,), jnp.int32),
            pltpu.VMEM((CH,), jnp.int32),
            pltpu.VMEM((CH,), jnp.int32),
            pltpu.VMEM((CH,), jnp.int32),
            pltpu.VMEM((CH, D), jnp.float32),
            pltpu.VMEM((CH, D), jnp.float32),
            pltpu.VMEM_SHARED((N_PAD, D), jnp.float32),
            pltpu.SemaphoreType.DMA,
            pltpu.SemaphoreType.DMA,
        ))


def _make_deg(W=D_DEG):
    mesh = plsc.VectorSubcoreMesh(core_axis_name="c", subcore_axis_name="s")

    def body(src_hbm, dst_hbm, deg_out, didx_a, didx_b, ones, acc):
        c = lax.axis_index("c")
        s = lax.axis_index("s")
        wid = c * NTILES + s

        _fill(ones, W, 0.0)
        base_r = s * RPT
        for j in range(RPT // CH):
            pltpu.sync_copy(ones, acc.at[pl.ds(base_r + j * CH, CH)])
        _fill(ones, W, 1.0)
        plsc.subcore_barrier()

        ebase = wid * EPW
        pltpu.sync_copy(dst_hbm.at[pl.ds(ebase, CH)], didx_a)

        def step(j, _):
            i0 = 2 * j
            pltpu.sync_copy(dst_hbm.at[pl.ds(ebase + (i0 + 1) * CH, CH)],
                            didx_b)
            pltpu.sync_copy(ones, acc.at[didx_a], add=True)
            pltpu.sync_copy(dst_hbm.at[pl.ds(ebase + (i0 + 2) * CH, CH)],
                            didx_a)
            pltpu.sync_copy(ones, acc.at[didx_b], add=True)
            return 0

        lax.fori_loop(0, NPAIR, step, 0)
        plsc.subcore_barrier()

        for j in range(RPT // CH):
            r0 = base_r + j * CH
            pltpu.sync_copy(acc.at[pl.ds(r0, CH)], ones)
            pltpu.sync_copy(ones, deg_out.at[pl.ds(c * N_PAD + r0, CH)])

    return pl.kernel(
        body,
        out_type=jax.ShapeDtypeStruct((NCORES * N_PAD, W), jnp.float32),
        mesh=mesh,
        scratch_types=(
            pltpu.VMEM((CH,), jnp.int32),
            pltpu.VMEM((CH,), jnp.int32),
            pltpu.VMEM((CH, W), jnp.float32),
            pltpu.VMEM_SHARED((N_PAD, W), jnp.float32),
        ))


_agg_h = _make_agg(D_H)
_deg_k = _make_deg()


def _pre_body(x_ref, wl_ref, wr_ref, bl_ref, p_ref, r_ref):
    h = x_ref[...]
    p_ref[...] = jnp.dot(h, wl_ref[...], preferred_element_type=jnp.float32)
    r_ref[...] = (jnp.dot(h, wr_ref[...], preferred_element_type=jnp.float32)
                  + bl_ref[...])


def _pre(x, wlT, wrT, bl):
    d_in, d_o = wlT.shape
    return pl.pallas_call(
        _pre_body,
        grid=(GRID,),
        in_specs=[
            pl.BlockSpec((BLK, d_in), lambda i: (i, 0)),
            pl.BlockSpec((d_in, d_o), lambda i: (0, 0)),
            pl.BlockSpec((d_in, d_o), lambda i: (0, 0)),
            pl.BlockSpec((1, d_o), lambda i: (0, 0)),
        ],
        out_specs=[
            pl.BlockSpec((BLK, d_o), lambda i: (i, 0)),
            pl.BlockSpec((BLK, d_o), lambda i: (i, 0)),
        ],
        out_shape=[
            jax.ShapeDtypeStruct((N_PAD, d_o), jnp.float32),
            jax.ShapeDtypeStruct((N_PAD, d_o), jnp.float32),
        ],
    )(x, wlT, wrT, bl)


def _combine(acc_ref, dacc_ref, r_ref):
    deg = dacc_ref[0] + dacc_ref[1]
    mean = (acc_ref[0] + acc_ref[1]) / jnp.maximum(deg, 1.0)
    return mean + r_ref[...]


def _mid_body(acc_ref, dacc_ref, r_ref, wl_ref, wr_ref, bl_ref, p_ref, rn_ref):
    h = jnp.maximum(_combine(acc_ref, dacc_ref, r_ref), 0.0)
    p_ref[...] = jnp.dot(h, wl_ref[...], preferred_element_type=jnp.float32)
    rn_ref[...] = (jnp.dot(h, wr_ref[...], preferred_element_type=jnp.float32)
                   + bl_ref[...])


def _mid(acc, dacc, r, wlT, wrT, bl):
    d, d_po = wlT.shape
    d_ro = wrT.shape[1]
    return pl.pallas_call(
        _mid_body,
        grid=(GRID,),
        in_specs=[
            pl.BlockSpec((NCORES, BLK, d), lambda i: (0, i, 0)),
            pl.BlockSpec((NCORES, BLK, 1), lambda i: (0, i, 0)),
            pl.BlockSpec((BLK, d), lambda i: (i, 0)),
            pl.BlockSpec((d, d_po), lambda i: (0, 0)),
            pl.BlockSpec((d, d_ro), lambda i: (0, 0)),
            pl.BlockSpec((1, d_ro), lambda i: (0, 0)),
        ],
        out_specs=[
            pl.BlockSpec((BLK, d_po), lambda i: (i, 0)),
            pl.BlockSpec((BLK, d_ro), lambda i: (i, 0)),
        ],
        out_shape=[
            jax.ShapeDtypeStruct((N_PAD, d_po), jnp.float32),
            jax.ShapeDtypeStruct((N_PAD, d_ro), jnp.float32),
        ],
    )(acc, dacc, r, wlT, wrT, bl)


def _final_body(acc_ref, dacc_ref, r_ref, o_ref):
    d = r_ref.shape[1]
    deg = dacc_ref[0] + dacc_ref[1]
    mean = (acc_ref[0, :, :d] + acc_ref[1, :, :d]) / jnp.maximum(deg, 1.0)
    z = mean + r_ref[...]
    m = jnp.max(z, axis=1, keepdims=True)
    ez = jnp.exp(z - m)
    lse = jnp.log(jnp.sum(ez, axis=1, keepdims=True)) + m
    o_ref[...] = z - lse


def _final(acc, dacc, r):
    d = r.shape[1]
    return pl.pallas_call(
        _final_body,
        grid=(GRID,),
        in_specs=[
            pl.BlockSpec((NCORES, BLK, acc.shape[2]), lambda i: (0, i, 0)),
            pl.BlockSpec((NCORES, BLK, 1), lambda i: (0, i, 0)),
            pl.BlockSpec((BLK, d), lambda i: (i, 0)),
        ],
        out_specs=pl.BlockSpec((BLK, d), lambda i: (i, 0)),
        out_shape=jax.ShapeDtypeStruct((N_PAD, d), jnp.float32),
    )(acc, dacc, r)


@jax.jit
def kernel(x, edge_index, Wl1, bl1, Wr1, Wl2, bl2, Wr2, Wl3, bl3, Wr3):
    x_pad = jnp.zeros((N_PAD, D_IN), jnp.float32).at[:N].set(x)
    pad_e = E_PAD + 2 * CH - E
    src = jnp.concatenate([edge_index[0], jnp.zeros((pad_e,), jnp.int32)])
    dst = jnp.concatenate([edge_index[1], jnp.full((pad_e,), N, jnp.int32)])

    dacc = _deg_k(src, dst).reshape(NCORES, N_PAD, D_DEG)[:, :, :1]

    p1, r1 = _pre(x_pad, Wl1.T, Wr1.T, bl1[None])
    acc1 = _agg_h(p1, src, dst).reshape(NCORES, N_PAD, D_H)
    p2, r2 = _mid(acc1, dacc, r1, Wl2.T, Wr2.T, bl2[None])
    acc2 = _agg_h(p2, src, dst).reshape(NCORES, N_PAD, D_H)
    wl3T_pad = jnp.pad(Wl3.T, ((0, 0), (0, D_H - D_OUT)))
    p3, r3 = _mid(acc2, dacc, r2, wl3T_pad, Wr3.T, bl3[None])
    acc3 = _agg_h(p3, src, dst).reshape(NCORES, N_PAD, D_H)
    out = _final(acc3, dacc, r3)
    return out[:N]
